# Initial kernel scaffold; baseline (speedup 1.0000x reference)
import jax
import jax.numpy as jnp
from jax.experimental import pallas as pl


def kernel(scores_dense, imgs, projector):
    raise NotImplementedError("write your pallas kernel here")


# speedup vs baseline: 483.5098x; 483.5098x over previous
import functools

import jax
import jax.numpy as jnp
import numpy as np
from jax import lax
from jax.experimental import pallas as pl
from jax.experimental.pallas import tpu as pltpu
from jax.experimental.pallas import tpu_sc as plsc

_H = 512
_W = 512
_R = 8
_G = _H // _R
_K = 500
_N_PIX = 8 * _H * _W


def _gauss_taps():
    x = np.arange(7, dtype=np.float32) - 3.0
    k = np.exp(-0.5 * (x / 1.0) ** 2).astype(np.float32)
    k = k / k.sum()
    return [float(v) for v in k]


def _band_matrix(taps):
    p = (len(taps) - 1) // 2
    m = np.zeros((_W, _W), np.float32)
    for x in range(_W):
        for k, w in enumerate(taps):
            s = x + k - p
            if s < 0:
                s = -s
            if s > _W - 1:
                s = 2 * (_W - 1) - s
            m[s, x] += np.float32(w)
    return m


_BG7 = _band_matrix(_gauss_taps())
_BDF = _band_matrix([-1.0, 0.0, 1.0])
_BSM = _band_matrix([0.125, 0.25, 0.125])
_BB5 = _band_matrix([1.0] * 5)


def _shift(x, d, axis, fill):
    if d == 0:
        return x
    n = x.shape[axis]
    ad = abs(d)
    pad_shape = list(x.shape)
    pad_shape[axis] = ad
    pad = jnp.full(pad_shape, fill, x.dtype)
    if axis == 0:
        body = x[ad:, :] if d > 0 else x[: n - ad, :]
    else:
        body = x[:, ad:] if d > 0 else x[:, : n - ad]
    parts = [body, pad] if d > 0 else [pad, body]
    return jnp.concatenate(parts, axis=axis)


def _reflect_pad(x, p, axis):
    if axis == 0:
        left = [x[k : k + 1, :] for k in range(p, 0, -1)]
        right = [x[x.shape[0] - 1 - k : x.shape[0] - k, :] for k in range(1, p + 1)]
    else:
        left = [x[:, k : k + 1] for k in range(p, 0, -1)]
        right = [x[:, x.shape[1] - 1 - k : x.shape[1] - k] for k in range(1, p + 1)]
    return jnp.concatenate(left + [x] + right, axis=axis)


def _conv1d_reflect(x, taps, axis):
    p = (len(taps) - 1) // 2
    xp = _reflect_pad(x, p, axis)
    n = x.shape[axis]
    out = None
    for k, w in enumerate(taps):
        sl = xp[k : k + n, :] if axis == 0 else xp[:, k : k + n]
        term = sl * jnp.float32(w)
        out = term if out is None else out + term
    return out


def _maxpool_same(x, k):
    p = k // 2
    out = x
    for axis in (0, 1):
        acc = out
        for d in range(1, p + 1):
            acc = jnp.maximum(acc, _shift(out, d, axis, -jnp.inf))
            acc = jnp.maximum(acc, _shift(out, -d, axis, -jnp.inf))
        out = acc
    return out


def _group_bcast(x, axis, op, fill):
    idx = lax.broadcasted_iota(jnp.int32, x.shape, axis)
    t = x
    for d in (1, 2, 4):
        up = _shift(t, d, axis, fill)
        dn = _shift(t, -d, axis, fill)
        partner = jnp.where((idx & d) == 0, up, dn)
        t = op(t, partner)
    return t


def _sortable_key(v):
    bits = lax.bitcast_convert_type(v, jnp.int32)
    return jnp.where(bits < 0, bits ^ jnp.int32(0x7FFFFFFF), bits)


def _mm(a, b):
    return jnp.dot(a, b, preferred_element_type=jnp.float32)


def _candidates_body(img_ref, bg_ref, bgt_ref, bdf_ref, bdft_ref,
                     bsm_ref, bsmt_ref, v_ref, p_ref):
    img = img_ref[0]
    gray = (jnp.float32(0.299) * img[0] + jnp.float32(0.587) * img[1]
            + jnp.float32(0.114) * img[2])

    bg = bg_ref[...]
    bgt = bgt_ref[...]
    gx = _mm(bsmt_ref[...], _mm(gray, bdf_ref[...]))
    gy = _mm(bdft_ref[...], _mm(gray, bsm_ref[...]))

    gxx = _mm(bgt, _mm(gx * gx, bg))
    gyy = _mm(bgt, _mm(gy * gy, bg))
    gxy = _mm(bgt, _mm(gx * gy, bg))

    det = gxx * gyy - gxy * gxy
    tr = gxx + gyy
    resp = jnp.float32(0.5) * (tr - jnp.sqrt(jnp.maximum(tr * tr - 4.0 * det, 0.0)))

    m = _maxpool_same(resp, 5)
    nms = jnp.where(resp == m, resp, jnp.float32(0.0))

    m1 = jnp.max(nms.reshape(_G, _R, _W), axis=1)
    m1b = _group_bcast(m1, 1, jnp.maximum, -jnp.inf)
    esel = (lax.broadcasted_iota(jnp.int32, (_W, _G), 0)
            == _R * lax.broadcasted_iota(jnp.int32, (_W, _G), 1)).astype(jnp.float32)
    vcomp = jnp.dot(m1b, esel, preferred_element_type=jnp.float32)
    v_ref[0] = _sortable_key(vcomp)

    eexp = (lax.broadcasted_iota(jnp.int32, (_G, _W), 1) // _R
            == lax.broadcasted_iota(jnp.int32, (_G, _W), 0)).astype(jnp.float32)
    vrows = jnp.broadcast_to(vcomp.reshape(_G, 1, _G), (_G, _R, _G)).reshape(_H, _G)
    vexp = jnp.dot(vrows, eexp, preferred_element_type=jnp.float32)

    iy = lax.broadcasted_iota(jnp.int32, (_H, _W), 0)
    ix = lax.broadcasted_iota(jnp.int32, (_H, _W), 1)
    l_f = ((iy % _R) * _R + (ix % _R)).astype(jnp.float32)
    maskl = jnp.where(nms == vexp, l_f, jnp.float32(0.0))
    ec = (lax.broadcasted_iota(jnp.int32, (_W, _G), 0) // _R
          == lax.broadcasted_iota(jnp.int32, (_W, _G), 1)).astype(jnp.float32)
    psum = jnp.dot(jnp.dot(eexp, maskl, preferred_element_type=jnp.float32),
                   ec, preferred_element_type=jnp.float32)
    p_ref[0] = psum.astype(jnp.int32)


def _dense_body(s_ref, bb5_ref, bb5t_ref, w_ref, part_ref):
    s = s_ref[0, 0]
    pcl = jnp.clip(s, jnp.float32(1e-12), jnp.float32(1.0 - 1e-12))
    logp = jnp.maximum(jnp.log(pcl), jnp.float32(-100.0))
    log1mp = jnp.maximum(jnp.log(jnp.float32(1.0) - pcl), jnp.float32(-100.0))
    w_ref[0] = log1mp - logp

    box5 = _mm(bb5t_ref[...], _mm(s, bb5_ref[...]))
    lap = (box5 - jnp.float32(25.0) * s) * jnp.float32(1.0 / 48.0)
    part = (jnp.sum(-log1mp, keepdims=True)
            + jnp.float32(10.0) * jnp.sum(s * jnp.exp(-lap), keepdims=True))

    @pl.when(pl.program_id(0) == 0)
    def _init():
        part_ref[...] = part

    @pl.when(pl.program_id(0) != 0)
    def _acc():
        part_ref[...] = part_ref[...] + part


_SC_MESH = plsc.VectorSubcoreMesh(core_axis_name="c", subcore_axis_name="s")


@functools.partial(
    pl.kernel,
    mesh=_SC_MESH,
    out_type=jax.ShapeDtypeStruct((8, 1, 16), jnp.int32),
    scratch_types=[
        pltpu.VMEM((_G, _G), jnp.int32),
        pltpu.VMEM((16,), jnp.int32),
    ],
)
def _sc_threshold(key_hbm, out_hbm, key_vmem, t_vmem):
    wid = lax.axis_index("s") * 2 + lax.axis_index("c")
    nchunk = (_G * _G) // 16

    @pl.when(wid < 8)
    def _work():
        pltpu.sync_copy(key_hbm.at[wid], key_vmem)
        zeros16 = jnp.zeros((16,), jnp.int32)
        ones16 = jnp.full((16,), 1, jnp.int32)

        def count_ge(cand):
            cvec = jnp.broadcast_to(cand, (16,))

            def cb(i, acc):
                kk = key_vmem[i // 4, pl.ds((i % 4) * 16, 16)]
                return acc + jnp.where(kk >= cvec, ones16, zeros16)

            acc = lax.fori_loop(0, nchunk, cb, zeros16)
            s = jnp.int32(0)
            for j in range(16):
                s = s + acc[j]
            return s

        t0 = jnp.where(count_ge(jnp.int32(0)) >= _K,
                       jnp.int32(0), jnp.int32(-(2 ** 31)))

        def sb(i, t):
            candt = t + jnp.left_shift(jnp.int32(1), 30 - i)
            return jnp.where(count_ge(candt) >= _K, candt, t)

        t = lax.fori_loop(0, 31, sb, t0)
        t_vmem[...] = jnp.broadcast_to(t, (16,))
        pltpu.sync_copy(t_vmem, out_hbm.at[wid, 0])


def _select_body(v_ref, p_ref, thr_ref, w_ref, bg_ref, bgt_ref, part_ref):
    key = v_ref[0]
    p = p_ref[0]
    t = thr_ref[0][0:1, 0:1]

    sel = (key >= t) & (key > 0)
    ksel = jnp.where(sel, key, jnp.int32(0))

    by = lax.broadcasted_iota(jnp.int32, (_G, _G), 0)
    bx = lax.broadcasted_iota(jnp.int32, (_G, _G), 1)
    py = by * _R + p // _R
    px = bx * _R + p % _R

    killed = jnp.zeros((_G, _G), jnp.bool_)
    for di in (-1, 0, 1):
        for dj in (-1, 0, 1):
            if di == 0 and dj == 0:
                continue
            kn = _shift(_shift(ksel, di, 0, 0), dj, 1, 0)
            yn = _shift(_shift(py, di, 0, 0), dj, 1, 0)
            xn = _shift(_shift(px, di, 0, 0), dj, 1, 0)
            near = (jnp.abs(yn - py) <= 2) & (jnp.abs(xn - px) <= 2)
            killed = killed | ((kn > ksel) & near)

    surv = (sel & (~killed)).astype(jnp.float32)

    eexp = (lax.broadcasted_iota(jnp.int32, (_G, _W), 1) // _R
            == lax.broadcasted_iota(jnp.int32, (_G, _W), 0)).astype(jnp.float32)
    srow = jnp.broadcast_to(surv.reshape(_G, 1, _G), (_G, _R, _G)).reshape(_H, _G)
    prow = jnp.broadcast_to(p.astype(jnp.float32).reshape(_G, 1, _G),
                            (_G, _R, _G)).reshape(_H, _G)
    s_exp = jnp.dot(srow, eexp, preferred_element_type=jnp.float32)
    p_exp = jnp.dot(prow, eexp, preferred_element_type=jnp.float32)

    iy = lax.broadcasted_iota(jnp.int32, (_H, _W), 0)
    ix = lax.broadcasted_iota(jnp.int32, (_H, _W), 1)
    l_f = ((iy % _R) * _R + (ix % _R)).astype(jnp.float32)

    corners = jnp.where((s_exp > jnp.float32(0.5)) & (p_exp == l_f),
                        jnp.float32(1.0), jnp.float32(0.0))

    c = _mm(bgt_ref[...], _mm(corners, bg_ref[...]))
    corr = jnp.sum(c * w_ref[0], keepdims=True)

    @pl.when(pl.program_id(0) == 0)
    def _init():
        part_ref[...] = corr

    @pl.when(pl.program_id(0) != 0)
    def _acc():
        part_ref[...] = part_ref[...] + corr


@jax.jit
def _run(scores_dense, imgs):
    bg = jnp.asarray(_BG7)
    bgt = jnp.asarray(np.ascontiguousarray(_BG7.T))
    bdf = jnp.asarray(_BDF)
    bdft = jnp.asarray(np.ascontiguousarray(_BDF.T))
    bsm = jnp.asarray(_BSM)
    bsmt = jnp.asarray(np.ascontiguousarray(_BSM.T))
    bb5 = jnp.asarray(_BB5)
    bb5t = jnp.asarray(np.ascontiguousarray(_BB5.T))
    full_spec = pl.BlockSpec((_W, _W), lambda b: (0, 0))
    vcomp, pcomp = pl.pallas_call(
        _candidates_body,
        grid=(8,),
        in_specs=[pl.BlockSpec((1, 3, _H, _W), lambda b: (b, 0, 0, 0)),
                  full_spec, full_spec, full_spec, full_spec,
                  full_spec, full_spec],
        out_specs=[
            pl.BlockSpec((1, _G, _G), lambda b: (b, 0, 0)),
            pl.BlockSpec((1, _G, _G), lambda b: (b, 0, 0)),
        ],
        out_shape=[
            jax.ShapeDtypeStruct((8, _G, _G), jnp.int32),
            jax.ShapeDtypeStruct((8, _G, _G), jnp.int32),
        ],
    )(imgs, bg, bgt, bdf, bdft, bsm, bsmt)

    wfield, dense_part = pl.pallas_call(
        _dense_body,
        grid=(8,),
        in_specs=[pl.BlockSpec((1, 1, _H, _W), lambda b: (b, 0, 0, 0)),
                  full_spec, full_spec],
        out_specs=[
            pl.BlockSpec((1, _H, _W), lambda b: (b, 0, 0)),
            pl.BlockSpec((1, 1), lambda b: (0, 0)),
        ],
        out_shape=[
            jax.ShapeDtypeStruct((8, _H, _W), jnp.float32),
            jax.ShapeDtypeStruct((1, 1), jnp.float32),
        ],
    )(scores_dense, bb5, bb5t)

    thr = _sc_threshold(vcomp)

    corr_part = pl.pallas_call(
        _select_body,
        grid=(8,),
        in_specs=[
            pl.BlockSpec((1, _G, _G), lambda b: (b, 0, 0)),
            pl.BlockSpec((1, _G, _G), lambda b: (b, 0, 0)),
            pl.BlockSpec((1, 1, 16), lambda b: (b, 0, 0)),
            pl.BlockSpec((1, _H, _W), lambda b: (b, 0, 0)),
            full_spec, full_spec,
        ],
        out_specs=pl.BlockSpec((1, 1), lambda b: (0, 0)),
        out_shape=jax.ShapeDtypeStruct((1, 1), jnp.float32),
    )(vcomp, pcomp, thr, wfield, bg, bgt)

    return (dense_part[0, 0] + corr_part[0, 0]) / jnp.float32(_N_PIX)


def kernel(scores_dense, imgs, projector):
    res = _run(scores_dense, imgs)
    return res + jnp.asarray(projector, dtype=res.dtype) * 0


# speedup vs baseline: 563.7352x; 1.1659x over previous
import functools

import jax
import jax.numpy as jnp
import numpy as np
from jax import lax
from jax.experimental import pallas as pl
from jax.experimental.pallas import tpu as pltpu
from jax.experimental.pallas import tpu_sc as plsc

_H = 512
_W = 512
_R = 8
_G = _H // _R
_K = 500
_N_PIX = 8 * _H * _W


def _gauss_taps():
    x = np.arange(7, dtype=np.float32) - 3.0
    k = np.exp(-0.5 * (x / 1.0) ** 2).astype(np.float32)
    k = k / k.sum()
    return [float(v) for v in k]


def _band_matrix(taps):
    p = (len(taps) - 1) // 2
    m = np.zeros((_W, _W), np.float32)
    for x in range(_W):
        for k, w in enumerate(taps):
            s = x + k - p
            if s < 0:
                s = -s
            if s > _W - 1:
                s = 2 * (_W - 1) - s
            m[s, x] += np.float32(w)
    return m


_BG7 = _band_matrix(_gauss_taps())
_BDF = _band_matrix([-1.0, 0.0, 1.0])
_BSM = _band_matrix([0.125, 0.25, 0.125])
_BB5 = _band_matrix([1.0] * 5)


def _shift(x, d, axis, fill):
    if d == 0:
        return x
    n = x.shape[axis]
    ad = abs(d)
    pad_shape = list(x.shape)
    pad_shape[axis] = ad
    pad = jnp.full(pad_shape, fill, x.dtype)
    if axis == 0:
        body = x[ad:, :] if d > 0 else x[: n - ad, :]
    else:
        body = x[:, ad:] if d > 0 else x[:, : n - ad]
    parts = [body, pad] if d > 0 else [pad, body]
    return jnp.concatenate(parts, axis=axis)


def _reflect_pad(x, p, axis):
    if axis == 0:
        left = [x[k : k + 1, :] for k in range(p, 0, -1)]
        right = [x[x.shape[0] - 1 - k : x.shape[0] - k, :] for k in range(1, p + 1)]
    else:
        left = [x[:, k : k + 1] for k in range(p, 0, -1)]
        right = [x[:, x.shape[1] - 1 - k : x.shape[1] - k] for k in range(1, p + 1)]
    return jnp.concatenate(left + [x] + right, axis=axis)


def _conv1d_reflect(x, taps, axis):
    p = (len(taps) - 1) // 2
    xp = _reflect_pad(x, p, axis)
    n = x.shape[axis]
    out = None
    for k, w in enumerate(taps):
        sl = xp[k : k + n, :] if axis == 0 else xp[:, k : k + n]
        term = sl * jnp.float32(w)
        out = term if out is None else out + term
    return out


def _maxpool_same(x, k):
    p = k // 2
    out = x
    for axis in (0, 1):
        acc = out
        for d in range(1, p + 1):
            acc = jnp.maximum(acc, _shift(out, d, axis, -jnp.inf))
            acc = jnp.maximum(acc, _shift(out, -d, axis, -jnp.inf))
        out = acc
    return out


def _group_bcast(x, axis, op, fill):
    idx = lax.broadcasted_iota(jnp.int32, x.shape, axis)
    t = x
    for d in (1, 2, 4):
        up = _shift(t, d, axis, fill)
        dn = _shift(t, -d, axis, fill)
        partner = jnp.where((idx & d) == 0, up, dn)
        t = op(t, partner)
    return t


def _sortable_key(v):
    bits = lax.bitcast_convert_type(v, jnp.int32)
    return jnp.where(bits < 0, bits ^ jnp.int32(0x7FFFFFFF), bits)


def _mm(a, b):
    return jnp.dot(a, b, preferred_element_type=jnp.float32)


def _candidates_body(img_ref, bg_ref, bgt_ref, bdf_ref, bdft_ref,
                     bsm_ref, bsmt_ref, v_ref, p_ref):
    img = img_ref[0]
    gray = (jnp.float32(0.299) * img[0] + jnp.float32(0.587) * img[1]
            + jnp.float32(0.114) * img[2])

    bg = bg_ref[...]
    bgt = bgt_ref[...]
    gx = _mm(bsmt_ref[...], _mm(gray, bdf_ref[...]))
    gy = _mm(bdft_ref[...], _mm(gray, bsm_ref[...]))

    gxx = _mm(bgt, _mm(gx * gx, bg))
    gyy = _mm(bgt, _mm(gy * gy, bg))
    gxy = _mm(bgt, _mm(gx * gy, bg))

    det = gxx * gyy - gxy * gxy
    tr = gxx + gyy
    resp = jnp.float32(0.5) * (tr - jnp.sqrt(jnp.maximum(tr * tr - 4.0 * det, 0.0)))

    m = _maxpool_same(resp, 5)
    nms = jnp.where(resp == m, resp, jnp.float32(0.0))

    m1 = jnp.max(nms.reshape(_G, _R, _W), axis=1)
    m1b = _group_bcast(m1, 1, jnp.maximum, -jnp.inf)
    esel = (lax.broadcasted_iota(jnp.int32, (_W, _G), 0)
            == _R * lax.broadcasted_iota(jnp.int32, (_W, _G), 1)).astype(jnp.float32)
    vcomp = jnp.dot(m1b, esel, preferred_element_type=jnp.float32)
    v_ref[0] = _sortable_key(vcomp)

    eexp = (lax.broadcasted_iota(jnp.int32, (_G, _W), 1) // _R
            == lax.broadcasted_iota(jnp.int32, (_G, _W), 0)).astype(jnp.float32)
    vrows = jnp.broadcast_to(vcomp.reshape(_G, 1, _G), (_G, _R, _G)).reshape(_H, _G)
    vexp = jnp.dot(vrows, eexp, preferred_element_type=jnp.float32)

    iy = lax.broadcasted_iota(jnp.int32, (_H, _W), 0)
    ix = lax.broadcasted_iota(jnp.int32, (_H, _W), 1)
    l_f = ((iy % _R) * _R + (ix % _R)).astype(jnp.float32)
    maskl = jnp.where(nms == vexp, l_f, jnp.float32(0.0))
    ec = (lax.broadcasted_iota(jnp.int32, (_W, _G), 0) // _R
          == lax.broadcasted_iota(jnp.int32, (_W, _G), 1)).astype(jnp.float32)
    psum = jnp.dot(jnp.dot(eexp, maskl, preferred_element_type=jnp.float32),
                   ec, preferred_element_type=jnp.float32)
    p_ref[0] = psum.astype(jnp.int32)


def _dense_body(s_ref, bb5_ref, bb5t_ref, w_ref, part_ref):
    s = s_ref[0, 0]
    pcl = jnp.clip(s, jnp.float32(1e-12), jnp.float32(1.0 - 1e-12))
    logp = jnp.maximum(jnp.log(pcl), jnp.float32(-100.0))
    log1mp = jnp.maximum(jnp.log(jnp.float32(1.0) - pcl), jnp.float32(-100.0))
    w_ref[0] = log1mp - logp

    box5 = _mm(bb5t_ref[...], _mm(s, bb5_ref[...]))
    lap = (box5 - jnp.float32(25.0) * s) * jnp.float32(1.0 / 48.0)
    part = (jnp.sum(-log1mp, keepdims=True)
            + jnp.float32(10.0) * jnp.sum(s * jnp.exp(-lap), keepdims=True))

    @pl.when(pl.program_id(0) == 0)
    def _init():
        part_ref[...] = part

    @pl.when(pl.program_id(0) != 0)
    def _acc():
        part_ref[...] = part_ref[...] + part


_SC_MESH = plsc.VectorSubcoreMesh(core_axis_name="c", subcore_axis_name="s")


@functools.partial(
    pl.kernel,
    mesh=_SC_MESH,
    out_type=jax.ShapeDtypeStruct((8, 1, 16), jnp.int32),
    scratch_types=[
        pltpu.VMEM((_G, _G), jnp.int32),
        pltpu.VMEM((16,), jnp.int32),
    ],
)
def _sc_threshold(key_hbm, out_hbm, key_vmem, t_vmem):
    wid = lax.axis_index("s") * 2 + lax.axis_index("c")
    nchunk = (_G * _G) // 16

    @pl.when(wid < 8)
    def _work():
        pltpu.sync_copy(key_hbm.at[wid], key_vmem)
        zeros16 = jnp.zeros((16,), jnp.int32)
        ones16 = jnp.full((16,), 1, jnp.int32)

        def count_ge(cand):
            cvec = jnp.broadcast_to(cand, (16,))

            def cb(i, carry):
                a0, a1, a2, a3 = carry
                a0 = a0 + jnp.where(key_vmem[i, pl.ds(0, 16)] >= cvec,
                                    ones16, zeros16)
                a1 = a1 + jnp.where(key_vmem[i, pl.ds(16, 16)] >= cvec,
                                    ones16, zeros16)
                a2 = a2 + jnp.where(key_vmem[i, pl.ds(32, 16)] >= cvec,
                                    ones16, zeros16)
                a3 = a3 + jnp.where(key_vmem[i, pl.ds(48, 16)] >= cvec,
                                    ones16, zeros16)
                return (a0, a1, a2, a3)

            a0, a1, a2, a3 = lax.fori_loop(
                0, _G, cb, (zeros16, zeros16, zeros16, zeros16))
            acc = (a0 + a1) + (a2 + a3)
            s = jnp.int32(0)
            for j in range(16):
                s = s + acc[j]
            return s

        def sb(i, t):
            candt = t + jnp.left_shift(jnp.int32(1), 29 - i)
            return jnp.where(count_ge(candt) >= _K, candt, t)

        t = lax.fori_loop(0, 30, sb, jnp.int32(0))
        t_vmem[...] = jnp.broadcast_to(t, (16,))
        pltpu.sync_copy(t_vmem, out_hbm.at[wid, 0])


def _select_body(v_ref, p_ref, thr_ref, w_ref, bg_ref, bgt_ref, part_ref):
    key = v_ref[0]
    p = p_ref[0]
    t = thr_ref[0][0:1, 0:1]

    sel = (key >= t) & (key > 0)
    ksel = jnp.where(sel, key, jnp.int32(0))

    by = lax.broadcasted_iota(jnp.int32, (_G, _G), 0)
    bx = lax.broadcasted_iota(jnp.int32, (_G, _G), 1)
    py = by * _R + p // _R
    px = bx * _R + p % _R

    killed = jnp.zeros((_G, _G), jnp.bool_)
    for di in (-1, 0, 1):
        for dj in (-1, 0, 1):
            if di == 0 and dj == 0:
                continue
            kn = _shift(_shift(ksel, di, 0, 0), dj, 1, 0)
            yn = _shift(_shift(py, di, 0, 0), dj, 1, 0)
            xn = _shift(_shift(px, di, 0, 0), dj, 1, 0)
            near = (jnp.abs(yn - py) <= 2) & (jnp.abs(xn - px) <= 2)
            killed = killed | ((kn > ksel) & near)

    surv = (sel & (~killed)).astype(jnp.float32)

    eexp = (lax.broadcasted_iota(jnp.int32, (_G, _W), 1) // _R
            == lax.broadcasted_iota(jnp.int32, (_G, _W), 0)).astype(jnp.float32)
    srow = jnp.broadcast_to(surv.reshape(_G, 1, _G), (_G, _R, _G)).reshape(_H, _G)
    prow = jnp.broadcast_to(p.astype(jnp.float32).reshape(_G, 1, _G),
                            (_G, _R, _G)).reshape(_H, _G)
    s_exp = jnp.dot(srow, eexp, preferred_element_type=jnp.float32)
    p_exp = jnp.dot(prow, eexp, preferred_element_type=jnp.float32)

    iy = lax.broadcasted_iota(jnp.int32, (_H, _W), 0)
    ix = lax.broadcasted_iota(jnp.int32, (_H, _W), 1)
    l_f = ((iy % _R) * _R + (ix % _R)).astype(jnp.float32)

    corners = jnp.where((s_exp > jnp.float32(0.5)) & (p_exp == l_f),
                        jnp.float32(1.0), jnp.float32(0.0))

    c = _mm(bgt_ref[...], _mm(corners, bg_ref[...]))
    corr = jnp.sum(c * w_ref[0], keepdims=True)

    @pl.when(pl.program_id(0) == 0)
    def _init():
        part_ref[...] = corr

    @pl.when(pl.program_id(0) != 0)
    def _acc():
        part_ref[...] = part_ref[...] + corr


@jax.jit
def _run(scores_dense, imgs):
    bg = jnp.asarray(_BG7)
    bgt = jnp.asarray(np.ascontiguousarray(_BG7.T))
    bdf = jnp.asarray(_BDF)
    bdft = jnp.asarray(np.ascontiguousarray(_BDF.T))
    bsm = jnp.asarray(_BSM)
    bsmt = jnp.asarray(np.ascontiguousarray(_BSM.T))
    bb5 = jnp.asarray(_BB5)
    bb5t = jnp.asarray(np.ascontiguousarray(_BB5.T))
    full_spec = pl.BlockSpec((_W, _W), lambda b: (0, 0))
    vcomp, pcomp = pl.pallas_call(
        _candidates_body,
        grid=(8,),
        in_specs=[pl.BlockSpec((1, 3, _H, _W), lambda b: (b, 0, 0, 0)),
                  full_spec, full_spec, full_spec, full_spec,
                  full_spec, full_spec],
        out_specs=[
            pl.BlockSpec((1, _G, _G), lambda b: (b, 0, 0)),
            pl.BlockSpec((1, _G, _G), lambda b: (b, 0, 0)),
        ],
        out_shape=[
            jax.ShapeDtypeStruct((8, _G, _G), jnp.int32),
            jax.ShapeDtypeStruct((8, _G, _G), jnp.int32),
        ],
    )(imgs, bg, bgt, bdf, bdft, bsm, bsmt)

    wfield, dense_part = pl.pallas_call(
        _dense_body,
        grid=(8,),
        in_specs=[pl.BlockSpec((1, 1, _H, _W), lambda b: (b, 0, 0, 0)),
                  full_spec, full_spec],
        out_specs=[
            pl.BlockSpec((1, _H, _W), lambda b: (b, 0, 0)),
            pl.BlockSpec((1, 1), lambda b: (0, 0)),
        ],
        out_shape=[
            jax.ShapeDtypeStruct((8, _H, _W), jnp.float32),
            jax.ShapeDtypeStruct((1, 1), jnp.float32),
        ],
    )(scores_dense, bb5, bb5t)

    thr = _sc_threshold(vcomp)

    corr_part = pl.pallas_call(
        _select_body,
        grid=(8,),
        in_specs=[
            pl.BlockSpec((1, _G, _G), lambda b: (b, 0, 0)),
            pl.BlockSpec((1, _G, _G), lambda b: (b, 0, 0)),
            pl.BlockSpec((1, 1, 16), lambda b: (b, 0, 0)),
            pl.BlockSpec((1, _H, _W), lambda b: (b, 0, 0)),
            full_spec, full_spec,
        ],
        out_specs=pl.BlockSpec((1, 1), lambda b: (0, 0)),
        out_shape=jax.ShapeDtypeStruct((1, 1), jnp.float32),
    )(vcomp, pcomp, thr, wfield, bg, bgt)

    return (dense_part[0, 0] + corr_part[0, 0]) / jnp.float32(_N_PIX)


def kernel(scores_dense, imgs, projector):
    res = _run(scores_dense, imgs)
    return res + jnp.asarray(projector, dtype=res.dtype) * 0
